# Initial kernel scaffold; baseline (speedup 1.0000x reference)
import jax
import jax.numpy as jnp
from jax.experimental import pallas as pl


def kernel(image_features, degrees, text_embed, degree_embedding, depth_embedding, ln_gamma, ln_beta):
    raise NotImplementedError("write your pallas kernel here")


# speedup vs baseline: 4.1018x; 4.1018x over previous
import functools

import jax
import jax.numpy as jnp
from jax.experimental import pallas as pl
from jax.experimental.pallas import tpu as pltpu

B, L, W = 1024, 200, 512
NROWS = B * L
ROWS_PER_BLOCK = 1024


def _tc_body(idx_ref, img_ref, degtab_ref, depth_ref, gamma_ref, beta_ref,
             out_ref):
    r = pl.program_id(0) * ROWS_PER_BLOCK
    idx = idx_ref[0, 0, :]
    oh = (idx[:, None] == jax.lax.broadcasted_iota(jnp.int32, (1, 32), 1))
    deg = jnp.dot(oh.astype(jnp.float32), degtab_ref[...],
                  preferred_element_type=jnp.float32)
    pos = (r + jax.lax.broadcasted_iota(jnp.int32, (ROWS_PER_BLOCK, 1), 0)) % L
    dsel = jnp.where(pos >= (L // 2), depth_ref[1][None, :],
                     depth_ref[0][None, :])
    x = img_ref[...] + deg + dsel
    mean = jnp.mean(x, axis=-1, keepdims=True)
    xc = x - mean
    var = jnp.mean(xc * xc, axis=-1, keepdims=True)
    y = xc * jax.lax.rsqrt(var + 1e-5)
    out_ref[...] = y * gamma_ref[...][None, :] + beta_ref[...][None, :]


@jax.jit
def kernel(image_features, degrees, text_embed, degree_embedding,
           depth_embedding, ln_gamma, ln_beta):
    del text_embed
    img = image_features.reshape(NROWS, W)
    nblk = NROWS // ROWS_PER_BLOCK
    idx3 = degrees.reshape(nblk, 1, ROWS_PER_BLOCK)
    degtab = jnp.zeros((32, W), jnp.float32).at[:30].set(degree_embedding)

    out = pl.pallas_call(
        _tc_body,
        grid=(nblk,),
        in_specs=[
            pl.BlockSpec((1, 1, ROWS_PER_BLOCK), lambda i: (i, 0, 0)),
            pl.BlockSpec((ROWS_PER_BLOCK, W), lambda i: (i, 0)),
            pl.BlockSpec((32, W), lambda i: (0, 0)),
            pl.BlockSpec((2, W), lambda i: (0, 0)),
            pl.BlockSpec((W,), lambda i: (0,)),
            pl.BlockSpec((W,), lambda i: (0,)),
        ],
        out_specs=pl.BlockSpec((ROWS_PER_BLOCK, W), lambda i: (i, 0)),
        out_shape=jax.ShapeDtypeStruct((NROWS, W), jnp.float32),
    )(idx3, img, degtab, depth_embedding, ln_gamma, ln_beta)
    return out.reshape(B, L, W)
